# baseline (speedup 1.0000x reference)
import functools

import jax
import jax.numpy as jnp
from jax import lax
from jax.experimental import pallas as pl
from jax.experimental.pallas import tpu as pltpu
from jax.experimental.pallas import tpu_sc as plsc

NC = 2
NS = 16
LANES = 16


def _project_kernel(wc_ref, wb_ref, x_ref, out_ref):
    r = pl.program_id(1)
    w = (wc_ref[r, 0] * wb_ref[0]
         + wc_ref[r, 1] * wb_ref[1]
         + wc_ref[r, 2] * wb_ref[2]
         + wc_ref[r, 3] * wb_ref[3])
    out_ref[0] = jnp.dot(x_ref[...], w, preferred_element_type=jnp.float32)


def _finalize_kernel(p_ref, out_ref):
    out_ref[...] = jnp.maximum(p_ref[0] + p_ref[1], 0.0)


def _sc_edge_kernel(n_nodes, n_acc, n_edges, d, chunk,
                    pre_hbm, e3_hbm, zeros_hbm, part_hbm,
                    e3a, e3b, dst_a, dst_b, idx_a, idx_b, rows_a, rows_b, acc,
                    gsem_a, gsem_b, esem_a, esem_b, ssem_a, ssem_b):
    c = lax.axis_index("c")
    s = lax.axis_index("s")
    wid = c * NS + s

    edges_per_tile = n_edges // (NC * NS)
    n_chunks = edges_per_tile // chunk

    base = wid * edges_per_tile
    e3s = (e3a, e3b)
    dsts = (dst_a, dst_b)
    idxs = (idx_a, idx_b)
    rows = (rows_a, rows_b)
    gsems = (gsem_a, gsem_b)
    esems = (esem_a, esem_b)
    ssems = (ssem_a, ssem_b)

    def eload(ch, b):
        off3 = (base + ch * chunk) * 3
        pltpu.async_copy(e3_hbm.at[pl.ds(off3, 3 * chunk)], e3s[b], esems[b])

    def ewait(ch, b):
        off3 = (base + ch * chunk) * 3
        pltpu.make_async_copy(e3_hbm.at[pl.ds(off3, 3 * chunk)], e3s[b],
                              esems[b]).wait()

    def sdrain(b):
        pltpu.make_async_copy(rows[b], acc.at[dsts[b]], ssems[b]).wait()

    def stage(ch, b, sguard):
        ewait(ch, b)
        if sguard is True:
            sdrain(b)
        elif sguard is not None:
            @pl.when(sguard)
            def _():
                sdrain(b)
        for i in range(chunk // LANES):
            sl = pl.ds(i * LANES, LANES)
            idxs[b][sl] = (e3s[b][pl.ds(chunk + i * LANES, LANES)] * n_nodes
                           + e3s[b][pl.ds(i * LANES, LANES)])
            dsts[b][sl] = e3s[b][pl.ds(2 * chunk + i * LANES, LANES)]
        pltpu.async_copy(pre_hbm.at[idxs[b]], rows[b], gsems[b])

    def finish(b):
        pltpu.make_async_copy(pre_hbm.at[idxs[b]], rows[b], gsems[b]).wait()
        pltpu.async_copy(rows[b], acc.at[dsts[b]], ssems[b], add=True)

    eload(0, 0)
    eload(1, 1)
    zrows = n_acc // NS
    pltpu.sync_copy(zeros_hbm, acc.at[pl.ds(s * zrows, zrows)])
    plsc.subcore_barrier()

    stage(0, 0, None)

    def body(p, _):
        ch = 2 * p
        eload(ch + 2, 0)
        stage(ch + 1, 1, p > 0)
        finish(0)

        @pl.when(p < (n_chunks - 3) // 2)
        def _():
            eload(ch + 3, 1)

        stage(ch + 2, 0, True)
        finish(1)
        return ()

    lax.fori_loop(0, (n_chunks - 1) // 2, body, (), unroll=False)
    finish(0)
    sdrain(0)
    sdrain(1)

    plsc.subcore_barrier()

    rblk = 80
    n_rblk = n_nodes // rblk

    def out_body(it, _):
        j = it * NS + s

        @pl.when(j < n_rblk)
        def _():
            pltpu.sync_copy(acc.at[pl.ds(j * rblk, rblk)],
                            part_hbm.at[c, pl.ds(j * rblk, rblk)])
        return ()

    lax.fori_loop(0, (n_rblk + NS - 1) // NS, out_body, ())


def kernel(x, edge_index, edge_type, W_basis, W_comp):
    n_nodes, d_in = x.shape
    n_basis, _, d_out = W_basis.shape
    n_rel = W_comp.shape[0]
    n_edges = edge_type.shape[0]

    src = edge_index[0].astype(jnp.int32)
    dst = edge_index[1].astype(jnp.int32)
    typ = edge_type.astype(jnp.int32)

    bn = 2000
    nb = n_nodes // bn
    pre = pl.pallas_call(
        _project_kernel,
        grid=(nb, n_rel),
        in_specs=[
            pl.BlockSpec(memory_space=pltpu.SMEM),
            pl.BlockSpec((n_basis, d_in, d_out), lambda b, r: (0, 0, 0)),
            pl.BlockSpec((bn, d_in), lambda b, r: (b, 0)),
        ],
        out_specs=pl.BlockSpec((1, bn, d_out), lambda b, r: (r, b, 0)),
        out_shape=jax.ShapeDtypeStruct((n_rel, n_nodes, d_out), jnp.float32),
    )(W_comp, W_basis, x)
    pre_flat = pre.reshape(n_rel * n_nodes, d_out)

    chunk = 80
    n_acc = -(-n_nodes // (NS * 8)) * NS * 8
    zeros = jnp.zeros((n_acc // NS, d_out), jnp.float32)
    e3 = jnp.stack([src.reshape(-1, chunk), typ.reshape(-1, chunk),
                    dst.reshape(-1, chunk)], axis=1).reshape(-1)

    mesh = plsc.VectorSubcoreMesh(core_axis_name="c", subcore_axis_name="s")
    sc_fn = pl.kernel(
        functools.partial(_sc_edge_kernel, n_nodes, n_acc, n_edges, d_out,
                          chunk),
        out_type=jax.ShapeDtypeStruct((NC, n_nodes, d_out), jnp.float32),
        mesh=mesh,
        scratch_types=[
            pltpu.VMEM((3 * chunk,), jnp.int32),
            pltpu.VMEM((3 * chunk,), jnp.int32),
            pltpu.VMEM((chunk,), jnp.int32),
            pltpu.VMEM((chunk,), jnp.int32),
            pltpu.VMEM((chunk,), jnp.int32),
            pltpu.VMEM((chunk,), jnp.int32),
            pltpu.VMEM((chunk, d_out), jnp.float32),
            pltpu.VMEM((chunk, d_out), jnp.float32),
            pltpu.VMEM_SHARED((n_acc, d_out), jnp.float32),
            pltpu.SemaphoreType.DMA,
            pltpu.SemaphoreType.DMA,
            pltpu.SemaphoreType.DMA,
            pltpu.SemaphoreType.DMA,
            pltpu.SemaphoreType.DMA,
            pltpu.SemaphoreType.DMA,
        ],
    )
    partials = sc_fn(pre_flat, e3, zeros)

    out = pl.pallas_call(
        _finalize_kernel,
        grid=(nb,),
        in_specs=[pl.BlockSpec((NC, bn, d_out), lambda b: (0, b, 0))],
        out_specs=pl.BlockSpec((bn, d_out), lambda b: (b, 0)),
        out_shape=jax.ShapeDtypeStruct((n_nodes, d_out), jnp.float32),
    )(partials)
    return out


# speedup vs baseline: 1.1072x; 1.1072x over previous
import functools

import jax
import jax.numpy as jnp
from jax import lax
from jax.experimental import pallas as pl
from jax.experimental.pallas import tpu as pltpu
from jax.experimental.pallas import tpu_sc as plsc

NC = 2
NS = 16
LANES = 16


def _project_kernel(wc_ref, wb_ref, x_ref, out_ref):
    r = pl.program_id(1)
    w = (wc_ref[r, 0] * wb_ref[0]
         + wc_ref[r, 1] * wb_ref[1]
         + wc_ref[r, 2] * wb_ref[2]
         + wc_ref[r, 3] * wb_ref[3])
    out_ref[0] = jnp.dot(x_ref[...], w, preferred_element_type=jnp.float32)


def _finalize_kernel(p_ref, out_ref):
    out_ref[...] = jnp.maximum(p_ref[0] + p_ref[1], 0.0)


def _sc_edge_kernel(n_nodes, n_acc, n_edges, d, chunk,
                    pre_hbm, e3_hbm, zeros_hbm, part_hbm,
                    e3a, e3b, e3c, dst_a, dst_b, dst_c, idx_a, idx_b, idx_c,
                    rows_a, rows_b, rows_c, acc,
                    gsem_a, gsem_b, gsem_c, esem_a, esem_b, esem_c,
                    ssem_a, ssem_b, ssem_c):
    c = lax.axis_index("c")
    s = lax.axis_index("s")
    wid = c * NS + s

    edges_per_tile = n_edges // (NC * NS)
    n_chunks = edges_per_tile // chunk

    base = wid * edges_per_tile
    e3s = (e3a, e3b, e3c)
    dsts = (dst_a, dst_b, dst_c)
    idxs = (idx_a, idx_b, idx_c)
    rows = (rows_a, rows_b, rows_c)
    gsems = (gsem_a, gsem_b, gsem_c)
    esems = (esem_a, esem_b, esem_c)
    ssems = (ssem_a, ssem_b, ssem_c)

    def eload(ch, b):
        off3 = (base + ch * chunk) * 3
        pltpu.async_copy(e3_hbm.at[pl.ds(off3, 3 * chunk)], e3s[b], esems[b])

    def ewait(ch, b):
        off3 = (base + ch * chunk) * 3
        pltpu.make_async_copy(e3_hbm.at[pl.ds(off3, 3 * chunk)], e3s[b],
                              esems[b]).wait()

    def sdrain(b):
        pltpu.make_async_copy(rows[b], acc.at[dsts[b]], ssems[b]).wait()

    def stage(ch, b, sguard):
        ewait(ch, b)
        if sguard is True:
            sdrain(b)
        elif sguard is not None:
            @pl.when(sguard)
            def _():
                sdrain(b)
        for i in range(chunk // LANES):
            sl = pl.ds(i * LANES, LANES)
            idxs[b][sl] = (e3s[b][pl.ds(chunk + i * LANES, LANES)] * n_nodes
                           + e3s[b][pl.ds(i * LANES, LANES)])
            dsts[b][sl] = e3s[b][pl.ds(2 * chunk + i * LANES, LANES)]
        pltpu.async_copy(pre_hbm.at[idxs[b]], rows[b], gsems[b])

    def finish(b):
        pltpu.make_async_copy(pre_hbm.at[idxs[b]], rows[b], gsems[b]).wait()
        pltpu.async_copy(rows[b], acc.at[dsts[b]], ssems[b], add=True)

    eload(0, 0)
    eload(1, 1)
    eload(2, 2)
    zrows = n_acc // NS
    pltpu.sync_copy(zeros_hbm, acc.at[pl.ds(s * zrows, zrows)])
    plsc.subcore_barrier()

    stage(0, 0, None)
    stage(1, 1, None)

    def body(p, _):
        ch = 3 * p
        eload(ch + 3, 0)
        stage(ch + 2, 2, p > 0)
        finish(0)
        eload(ch + 4, 1)
        stage(ch + 3, 0, True)
        finish(1)

        @pl.when(p < (n_chunks - 5) // 3)
        def _():
            eload(ch + 5, 2)

        stage(ch + 4, 1, True)
        finish(2)
        return ()

    lax.fori_loop(0, (n_chunks - 2) // 3, body, (), unroll=False)
    finish(0)
    finish(1)
    sdrain(0)
    sdrain(1)
    sdrain(2)

    plsc.subcore_barrier()

    rblk = 80
    n_rblk = n_nodes // rblk

    def out_body(it, _):
        j = it * NS + s

        @pl.when(j < n_rblk)
        def _():
            pltpu.sync_copy(acc.at[pl.ds(j * rblk, rblk)],
                            part_hbm.at[c, pl.ds(j * rblk, rblk)])
        return ()

    lax.fori_loop(0, (n_rblk + NS - 1) // NS, out_body, ())


def kernel(x, edge_index, edge_type, W_basis, W_comp):
    n_nodes, d_in = x.shape
    n_basis, _, d_out = W_basis.shape
    n_rel = W_comp.shape[0]
    n_edges = edge_type.shape[0]

    src = edge_index[0].astype(jnp.int32)
    dst = edge_index[1].astype(jnp.int32)
    typ = edge_type.astype(jnp.int32)

    bn = 2000
    nb = n_nodes // bn
    pre = pl.pallas_call(
        _project_kernel,
        grid=(nb, n_rel),
        in_specs=[
            pl.BlockSpec(memory_space=pltpu.SMEM),
            pl.BlockSpec((n_basis, d_in, d_out), lambda b, r: (0, 0, 0)),
            pl.BlockSpec((bn, d_in), lambda b, r: (b, 0)),
        ],
        out_specs=pl.BlockSpec((1, bn, d_out), lambda b, r: (r, b, 0)),
        out_shape=jax.ShapeDtypeStruct((n_rel, n_nodes, d_out), jnp.float32),
    )(W_comp, W_basis, x)
    pre_flat = pre.reshape(n_rel * n_nodes, d_out)

    chunk = 80
    n_acc = -(-n_nodes // (NS * 8)) * NS * 8
    zeros = jnp.zeros((n_acc // NS, d_out), jnp.float32)
    e3 = jnp.stack([src.reshape(-1, chunk), typ.reshape(-1, chunk),
                    dst.reshape(-1, chunk)], axis=1).reshape(-1)

    mesh = plsc.VectorSubcoreMesh(core_axis_name="c", subcore_axis_name="s")
    sc_fn = pl.kernel(
        functools.partial(_sc_edge_kernel, n_nodes, n_acc, n_edges, d_out,
                          chunk),
        out_type=jax.ShapeDtypeStruct((NC, n_nodes, d_out), jnp.float32),
        mesh=mesh,
        scratch_types=(
            [pltpu.VMEM((3 * chunk,), jnp.int32)] * 3
            + [pltpu.VMEM((chunk,), jnp.int32)] * 6
            + [pltpu.VMEM((chunk, d_out), jnp.float32)] * 3
            + [pltpu.VMEM_SHARED((n_acc, d_out), jnp.float32)]
            + [pltpu.SemaphoreType.DMA] * 9
        ),
    )
    partials = sc_fn(pre_flat, e3, zeros)

    out = pl.pallas_call(
        _finalize_kernel,
        grid=(nb,),
        in_specs=[pl.BlockSpec((NC, bn, d_out), lambda b: (0, b, 0))],
        out_specs=pl.BlockSpec((bn, d_out), lambda b: (b, 0)),
        out_shape=jax.ShapeDtypeStruct((n_nodes, d_out), jnp.float32),
    )(partials)
    return out


# speedup vs baseline: 1.1229x; 1.0142x over previous
import functools

import jax
import jax.numpy as jnp
from jax import lax
from jax.experimental import pallas as pl
from jax.experimental.pallas import tpu as pltpu
from jax.experimental.pallas import tpu_sc as plsc

NC = 2
NS = 16
LANES = 16


def _project_kernel(wc_ref, wb_ref, x_ref, out_ref):
    r = pl.program_id(1)
    w = (wc_ref[r, 0] * wb_ref[0]
         + wc_ref[r, 1] * wb_ref[1]
         + wc_ref[r, 2] * wb_ref[2]
         + wc_ref[r, 3] * wb_ref[3])
    out_ref[0] = jnp.dot(x_ref[...], w, preferred_element_type=jnp.float32)


def _finalize_kernel(p_ref, out_ref):
    out_ref[...] = jnp.maximum(p_ref[0] + p_ref[1], 0.0)


def _sc_edge_kernel(n_nodes, n_acc, n_edges, d, chunk, *refs):
    (pre_hbm, e3_hbm, zeros_hbm, part_hbm) = refs[:4]
    e3s = refs[4:8]
    dsts = refs[8:12]
    idxs = refs[12:16]
    rows = refs[16:20]
    acc = refs[20]
    gsems = refs[21:25]
    esems = refs[25:29]
    ssems = refs[29:33]
    c = lax.axis_index("c")
    s = lax.axis_index("s")
    wid = c * NS + s

    edges_per_tile = n_edges // (NC * NS)
    n_chunks = edges_per_tile // chunk

    base = wid * edges_per_tile

    def eload(ch, b):
        off3 = (base + ch * chunk) * 3
        pltpu.async_copy(e3_hbm.at[pl.ds(off3, 3 * chunk)], e3s[b], esems[b])

    def ewait(ch, b):
        off3 = (base + ch * chunk) * 3
        pltpu.make_async_copy(e3_hbm.at[pl.ds(off3, 3 * chunk)], e3s[b],
                              esems[b]).wait()

    def sdrain(b):
        pltpu.make_async_copy(rows[b], acc.at[dsts[b]], ssems[b]).wait()

    def stage(ch, b, sguard):
        ewait(ch, b)
        if sguard is True:
            sdrain(b)
        elif sguard is not None:
            @pl.when(sguard)
            def _():
                sdrain(b)
        for i in range(chunk // LANES):
            sl = pl.ds(i * LANES, LANES)
            idxs[b][sl] = (e3s[b][pl.ds(chunk + i * LANES, LANES)] * n_nodes
                           + e3s[b][pl.ds(i * LANES, LANES)])
            dsts[b][sl] = e3s[b][pl.ds(2 * chunk + i * LANES, LANES)]
        pltpu.async_copy(pre_hbm.at[idxs[b]], rows[b], gsems[b])

    def finish(b):
        pltpu.make_async_copy(pre_hbm.at[idxs[b]], rows[b], gsems[b]).wait()
        pltpu.async_copy(rows[b], acc.at[dsts[b]], ssems[b], add=True)

    for b in range(4):
        eload(b, b)
    zrows = n_acc // NS
    pltpu.sync_copy(zeros_hbm, acc.at[pl.ds(s * zrows, zrows)])
    plsc.subcore_barrier()

    stage(0, 0, None)
    stage(1, 1, None)
    stage(2, 2, None)
    n_full = (n_chunks - 5) // 4

    def body(p, _):
        ch = 4 * p
        for k in range(4):
            eload(ch + k + 4, k)
            stage(ch + k + 3, (k + 3) % 4, p > 0 if k == 0 else True)
            finish(k)
        return ()

    lax.fori_loop(0, n_full, body, (), unroll=False)
    t = 4 * n_full
    eload(t + 4, (t + 4) % 4)
    stage(t + 3, (t + 3) % 4, True if n_full else None)
    finish(t % 4)
    stage(t + 4, (t + 4) % 4, True)
    finish((t + 1) % 4)
    finish((t + 2) % 4)
    finish((t + 3) % 4)
    finish((t + 4) % 4)
    for b in range(4):
        sdrain(b)

    plsc.subcore_barrier()

    rblk = 80
    n_rblk = n_nodes // rblk

    def out_body(it, _):
        j = it * NS + s

        @pl.when(j < n_rblk)
        def _():
            pltpu.sync_copy(acc.at[pl.ds(j * rblk, rblk)],
                            part_hbm.at[c, pl.ds(j * rblk, rblk)])
        return ()

    lax.fori_loop(0, (n_rblk + NS - 1) // NS, out_body, ())


def kernel(x, edge_index, edge_type, W_basis, W_comp):
    n_nodes, d_in = x.shape
    n_basis, _, d_out = W_basis.shape
    n_rel = W_comp.shape[0]
    n_edges = edge_type.shape[0]

    src = edge_index[0].astype(jnp.int32)
    dst = edge_index[1].astype(jnp.int32)
    typ = edge_type.astype(jnp.int32)

    bn = 2000
    nb = n_nodes // bn
    pre = pl.pallas_call(
        _project_kernel,
        grid=(nb, n_rel),
        in_specs=[
            pl.BlockSpec(memory_space=pltpu.SMEM),
            pl.BlockSpec((n_basis, d_in, d_out), lambda b, r: (0, 0, 0)),
            pl.BlockSpec((bn, d_in), lambda b, r: (b, 0)),
        ],
        out_specs=pl.BlockSpec((1, bn, d_out), lambda b, r: (r, b, 0)),
        out_shape=jax.ShapeDtypeStruct((n_rel, n_nodes, d_out), jnp.float32),
    )(W_comp, W_basis, x)
    pre_flat = pre.reshape(n_rel * n_nodes, d_out)

    chunk = 80
    n_acc = -(-n_nodes // (NS * 8)) * NS * 8
    zeros = jnp.zeros((n_acc // NS, d_out), jnp.float32)
    e3 = jnp.stack([src.reshape(-1, chunk), typ.reshape(-1, chunk),
                    dst.reshape(-1, chunk)], axis=1).reshape(-1)

    mesh = plsc.VectorSubcoreMesh(core_axis_name="c", subcore_axis_name="s")
    sc_fn = pl.kernel(
        functools.partial(_sc_edge_kernel, n_nodes, n_acc, n_edges, d_out,
                          chunk),
        out_type=jax.ShapeDtypeStruct((NC, n_nodes, d_out), jnp.float32),
        mesh=mesh,
        scratch_types=(
            [pltpu.VMEM((3 * chunk,), jnp.int32)] * 4
            + [pltpu.VMEM((chunk,), jnp.int32)] * 8
            + [pltpu.VMEM((chunk, d_out), jnp.float32)] * 4
            + [pltpu.VMEM_SHARED((n_acc, d_out), jnp.float32)]
            + [pltpu.SemaphoreType.DMA] * 12
        ),
    )
    partials = sc_fn(pre_flat, e3, zeros)

    out = pl.pallas_call(
        _finalize_kernel,
        grid=(nb,),
        in_specs=[pl.BlockSpec((NC, bn, d_out), lambda b: (0, b, 0))],
        out_specs=pl.BlockSpec((bn, d_out), lambda b: (b, 0)),
        out_shape=jax.ShapeDtypeStruct((n_nodes, d_out), jnp.float32),
    )(partials)
    return out
